# Initial kernel scaffold; baseline (speedup 1.0000x reference)
import functools

import jax
import jax.numpy as jnp
from jax import lax
from jax.experimental import pallas as pl
from jax.experimental.pallas import tpu as pltpu
from jax.experimental.pallas import tpu_sc as plsc

NC = 2
NS = 16
NW = NC * NS
B = 16384
D = 16
NT = 26
BW = B // NW


def _body(*refs):
    cat_refs = refs[0:NT]
    table_refs = refs[NT:2 * NT]
    out_refs = refs[2 * NT:3 * NT]
    idx_v, rows_v, sem = refs[3 * NT:]

    wid = lax.axis_index("s") * NC + lax.axis_index("c")
    base = wid * BW

    for t in range(NT):
        pltpu.sync_copy(cat_refs[t].at[pl.ds(base, BW)], idx_v)
        pltpu.async_copy(table_refs[t].at[idx_v], rows_v, sem).wait()
        pltpu.sync_copy(rows_v, out_refs[t].at[pl.ds(base, BW)])


@jax.jit
def _embed_all(cats, tables):
    mesh = plsc.VectorSubcoreMesh(
        core_axis_name="c", subcore_axis_name="s",
        num_cores=NC, num_subcores=NS,
    )
    out_type = tuple(
        jax.ShapeDtypeStruct((B, D), jnp.float32) for _ in range(NT)
    )
    fn = pl.kernel(
        _body,
        out_type=out_type,
        mesh=mesh,
        scratch_types=[
            pltpu.VMEM((BW,), jnp.int32),
            pltpu.VMEM((BW, D), jnp.float32),
            pltpu.SemaphoreType.DMA,
        ],
    )
    return fn(*cats, *tables)


def kernel(cat_0, table_0, cat_1, table_1, cat_2, table_2, cat_3, table_3, cat_4, table_4, cat_5, table_5, cat_6, table_6, cat_7, table_7, cat_8, table_8, cat_9, table_9, cat_10, table_10, cat_11, table_11, cat_12, table_12, cat_13, table_13, cat_14, table_14, cat_15, table_15, cat_16, table_16, cat_17, table_17, cat_18, table_18, cat_19, table_19, cat_20, table_20, cat_21, table_21, cat_22, table_22, cat_23, table_23, cat_24, table_24, cat_25, table_25):
    args = locals()
    cats = tuple(args[f"cat_{i}"] for i in range(NT))
    tables = tuple(args[f"table_{i}"] for i in range(NT))
    return _embed_all(cats, tables)


# speedup vs baseline: 1.1095x; 1.1095x over previous
import functools

import jax
import jax.numpy as jnp
from jax import lax
from jax.experimental import pallas as pl
from jax.experimental.pallas import tpu as pltpu
from jax.experimental.pallas import tpu_sc as plsc

NC = 2
NS = 16
NW = NC * NS
B = 16384
D = 16
NT = 26
BW = B // NW


def _body(*refs):
    cat_refs = refs[0:NT]
    table_refs = refs[NT:2 * NT]
    out_refs = refs[2 * NT:3 * NT]
    idx_v, rows_v, sem = refs[3 * NT:]

    wid = lax.axis_index("s") * NC + lax.axis_index("c")
    base = wid * BW

    for t in range(NT):
        pltpu.sync_copy(cat_refs[t].at[pl.ds(base, BW)], idx_v)
        pltpu.async_copy(table_refs[t].at[idx_v], rows_v, sem).wait()
        pltpu.sync_copy(rows_v, out_refs[t].at[pl.ds(base, BW)])


@jax.jit
def _embed_all(cats, tables):
    mesh = plsc.VectorSubcoreMesh(
        core_axis_name="c", subcore_axis_name="s",
        num_cores=NC, num_subcores=NS,
    )
    out_type = tuple(
        jax.ShapeDtypeStruct((B, D), jnp.float32) for _ in range(NT)
    )
    fn = pl.kernel(
        _body,
        out_type=out_type,
        mesh=mesh,
        scratch_types=[
            pltpu.VMEM((BW,), jnp.int32),
            pltpu.VMEM((BW, D), jnp.float32),
            pltpu.SemaphoreType.DMA,
        ],
        compiler_params=pltpu.CompilerParams(use_tc_tiling_on_sc=False),
    )
    return fn(*cats, *tables)


def kernel(cat_0, table_0, cat_1, table_1, cat_2, table_2, cat_3, table_3, cat_4, table_4, cat_5, table_5, cat_6, table_6, cat_7, table_7, cat_8, table_8, cat_9, table_9, cat_10, table_10, cat_11, table_11, cat_12, table_12, cat_13, table_13, cat_14, table_14, cat_15, table_15, cat_16, table_16, cat_17, table_17, cat_18, table_18, cat_19, table_19, cat_20, table_20, cat_21, table_21, cat_22, table_22, cat_23, table_23, cat_24, table_24, cat_25, table_25):
    args = locals()
    cats = tuple(args[f"cat_{i}"] for i in range(NT))
    tables = tuple(args[f"table_{i}"] for i in range(NT))
    return _embed_all(cats, tables)


# speedup vs baseline: 1.1416x; 1.0290x over previous
import functools

import jax
import jax.numpy as jnp
from jax import lax
from jax.experimental import pallas as pl
from jax.experimental.pallas import tpu as pltpu
from jax.experimental.pallas import tpu_sc as plsc

NC = 2
NS = 16
NW = NC * NS
B = 16384
D = 16
NT = 26
BW = B // NW


NB = 8
LA = 6


def _body(*refs):
    cat_refs = refs[0:NT]
    table_refs = refs[NT:2 * NT]
    out_refs = refs[2 * NT:3 * NT]
    idx_v = refs[3 * NT]
    rows = refs[3 * NT + 1:3 * NT + 1 + NB]
    sem_i = refs[3 * NT + 1 + NB]
    sem_g = refs[3 * NT + 2 + NB:3 * NT + 2 + 2 * NB]
    sem_s = refs[3 * NT + 2 + 2 * NB:3 * NT + 2 + 3 * NB]

    wid = lax.axis_index("s") * NC + lax.axis_index("c")
    base = wid * BW

    idx_copies = [
        pltpu.async_copy(cat_refs[t].at[pl.ds(base, BW)], idx_v.at[t], sem_i)
        for t in range(NT)
    ]
    for c in idx_copies:
        c.wait()

    def gather(t):
        b = t % NB
        return pltpu.async_copy(
            table_refs[t].at[idx_v.at[t]], rows[b], sem_g[b])

    def store(t):
        b = t % NB
        return pltpu.async_copy(
            rows[b], out_refs[t].at[pl.ds(base, BW)], sem_s[b])

    g = {}
    s = {}
    for t in range(min(LA, NT)):
        g[t] = gather(t)
    for t in range(NT):
        g[t].wait()
        s[t] = store(t)
        nxt = t + LA
        if nxt < NT:
            if nxt - NB >= 0:
                s[nxt - NB].wait()
            g[nxt] = gather(nxt)
    for t in range(max(0, NT - NB), NT):
        s[t].wait()


@jax.jit
def _embed_all(cats, tables):
    mesh = plsc.VectorSubcoreMesh(
        core_axis_name="c", subcore_axis_name="s",
        num_cores=NC, num_subcores=NS,
    )
    out_type = tuple(
        jax.ShapeDtypeStruct((B, D), jnp.float32) for _ in range(NT)
    )
    fn = pl.kernel(
        _body,
        out_type=out_type,
        mesh=mesh,
        scratch_types=(
            [pltpu.VMEM((NT, BW), jnp.int32)]
            + [pltpu.VMEM((BW, D), jnp.float32) for _ in range(NB)]
            + [pltpu.SemaphoreType.DMA for _ in range(1 + 2 * NB)]
        ),
        compiler_params=pltpu.CompilerParams(use_tc_tiling_on_sc=False),
    )
    return fn(*cats, *tables)


def kernel(cat_0, table_0, cat_1, table_1, cat_2, table_2, cat_3, table_3, cat_4, table_4, cat_5, table_5, cat_6, table_6, cat_7, table_7, cat_8, table_8, cat_9, table_9, cat_10, table_10, cat_11, table_11, cat_12, table_12, cat_13, table_13, cat_14, table_14, cat_15, table_15, cat_16, table_16, cat_17, table_17, cat_18, table_18, cat_19, table_19, cat_20, table_20, cat_21, table_21, cat_22, table_22, cat_23, table_23, cat_24, table_24, cat_25, table_25):
    args = locals()
    cats = tuple(args[f"cat_{i}"] for i in range(NT))
    tables = tuple(args[f"table_{i}"] for i in range(NT))
    return _embed_all(cats, tables)
